# baseline (speedup 1.0000x reference)
import functools

import jax
import jax.numpy as jnp
from jax import lax
from jax.experimental import pallas as pl
from jax.experimental.pallas import tpu as pltpu
from jax.experimental.pallas import tpu_sc as plsc

N = 10000
E = 160000
D = 300
L = 5
NSEG = 4
HW = 80
PAD1 = NSEG * HW - D

NC = 2
NS = 16
K = 80
CPT = E // NS
NCHUNK = CPT // K
STRIPE = N // NS
CPASS = 125
NPASS = STRIPE // CPASS
CPW = HW // 16
NDB = 5
NIB = 10
UNROLL = 10
NGRP = (NCHUNK + UNROLL - 1) // UNROLL


def _sc_aggr_body(h_hbm, ee_hbm, src_hbm, dst_hbm, out_hbm,
                  srcv, dstv, rows, eev, obuf, aggr_sh,
                  gsem, esem, ssem, isem):
    c = lax.axis_index("c")
    s = lax.axis_index("s")

    for p in range(2):
        q = 2 * c + p

        zv = jnp.zeros((16,), jnp.float32)

        def zrow(r, carry):
            for j in range(CPW):
                obuf[r, pl.ds(j * 16, 16)] = zv
            return carry

        lax.fori_loop(0, CPASS, zrow, 0)
        for pp in range(NPASS):
            pltpu.sync_copy(obuf,
                            aggr_sh.at[pl.ds(s * STRIPE + pp * CPASS, CPASS)])
        plsc.subcore_barrier()

        def idx_load(i, slot):
            base = s * CPT + i * K
            pltpu.async_copy(src_hbm.at[pl.ds(q * E + base, K)], srcv[slot],
                             isem[slot])
            pltpu.async_copy(dst_hbm.at[pl.ds(base, K)], dstv[slot],
                             isem[slot])

        def idx_wait(i, slot):
            base = s * CPT + i * K
            pltpu.make_async_copy(src_hbm.at[pl.ds(q * E + base, K)],
                                  srcv[slot], isem[slot]).wait()
            pltpu.make_async_copy(dst_hbm.at[pl.ds(base, K)], dstv[slot],
                                  isem[slot]).wait()

        def in_start(i, d, slot):
            pltpu.async_copy(h_hbm.at[srcv[slot]], rows[d], gsem[d])
            base = s * CPT + i * K
            pltpu.async_copy(ee_hbm.at[pl.ds(q * E + base, K)], eev[d],
                             esem[d])

        for i0 in range(5):
            idx_load(i0, i0)
        for i0 in range(3):
            idx_wait(i0, i0)
            in_start(i0, i0, i0)

        def group(g, carry):
            for j in range(UNROLL):
                i = g * UNROLL + j
                d4 = j % NDB
                d4n = (j + 3) % NDB
                i8 = j % NIB
                i8n = (j + 3) % NIB
                i8nn = (j + 5) % NIB
                live = i < NCHUNK

                @pl.when(live)
                def _():
                    pltpu.make_async_copy(h_hbm.at[srcv[i8]], rows[d4],
                                          gsem[d4]).wait()
                    base = s * CPT + i * K
                    pltpu.make_async_copy(
                        ee_hbm.at[pl.ds(q * E + base, K)], eev[d4],
                        esem[d4]).wait()

                    @plsc.parallel_loop(0, K, unroll=2)
                    def krow(k):
                        for jj in range(CPW):
                            sl = pl.ds(jj * 16, 16)
                            eev[d4][k, sl] = jnp.maximum(
                                rows[d4][k, sl] + eev[d4][k, sl], 0.0)

                    pltpu.async_copy(eev[d4], aggr_sh.at[dstv[i8]],
                                     ssem[d4], add=True)

                @pl.when((i >= 2) & (i - 2 < NCHUNK))
                def _():
                    pltpu.make_async_copy(eev[d4n],
                                          aggr_sh.at[dstv[(j + 8) % NIB]],
                                          ssem[d4n]).wait()

                @pl.when(i + 3 < NCHUNK)
                def _():
                    idx_wait(i + 3, i8n)
                    in_start(i + 3, d4n, i8n)

                @pl.when(i + 5 < NCHUNK)
                def _():
                    idx_load(i + 5, i8nn)
            return carry

        lax.fori_loop(0, NGRP, group, 0)
        plsc.subcore_barrier()

        for pp in range(NPASS):
            ro = s * STRIPE + pp * CPASS
            pltpu.sync_copy(aggr_sh.at[pl.ds(ro, CPASS)], obuf)
            pltpu.sync_copy(obuf, out_hbm.at[pl.ds(q * N + ro, CPASS)])
        plsc.subcore_barrier()


def _make_sc_aggr():
    mesh = plsc.VectorSubcoreMesh(core_axis_name="c", subcore_axis_name="s",
                                  num_cores=NC, num_subcores=NS)
    return pl.kernel(
        _sc_aggr_body,
        out_type=jax.ShapeDtypeStruct((NSEG * N, HW), jnp.float32),
        mesh=mesh,
        scratch_types=[
            [pltpu.VMEM((K,), jnp.int32) for _ in range(NIB)],
            [pltpu.VMEM((K,), jnp.int32) for _ in range(NIB)],
            [pltpu.VMEM((K, HW), jnp.float32) for _ in range(NDB)],
            [pltpu.VMEM((K, HW), jnp.float32) for _ in range(NDB)],
            pltpu.VMEM((CPASS, HW), jnp.float32),
            pltpu.VMEM_SHARED((N, HW), jnp.float32),
            [pltpu.SemaphoreType.DMA for _ in range(NDB)],
            [pltpu.SemaphoreType.DMA for _ in range(NDB)],
            [pltpu.SemaphoreType.DMA for _ in range(NDB)],
            [pltpu.SemaphoreType.DMA for _ in range(NIB)],
        ],
        compiler_params=pltpu.CompilerParams(use_tc_tiling_on_sc=False),
    )


BE = 2000
BN = 2000
NBN = N // BN


def _pack_segs(y, rows, out_ref):
    for qq in range(NSEG - 1):
        out_ref[qq] = y[:, qq * HW:(qq + 1) * HW]
    out_ref[NSEG - 1] = jnp.concatenate(
        [y[:, (NSEG - 1) * HW:], jnp.zeros((rows, PAD1), jnp.float32)], axis=1)


def _unpack(ref):
    return jnp.concatenate(
        [ref[qq] for qq in range(NSEG - 1)] + [ref[NSEG - 1][:, :HW - PAD1]],
        axis=1)


def _edge_enc_body(ea_ref, w1_ref, b1_ref, w2_ref, b2_ref, out_ref):
    ea = ea_ref[...]
    w1 = w1_ref[...]
    y = (ea[:, 0:1] * w1[0:1, :] + ea[:, 1:2] * w1[1:2, :]
         + ea[:, 2:3] * w1[2:3, :] + b1_ref[...])
    y = jnp.maximum(y, 0.0)
    y = jnp.dot(y, w2_ref[...], preferred_element_type=jnp.float32) + b2_ref[...]
    _pack_segs(y, BE, out_ref)


def _node_body(z_ref, ch_ref, fc_ref, tab_ref, wa1_ref, ba1_ref, wa2_ref,
               ba2_ref, out_ref):
    z = z_ref[...]
    ids = lax.broadcasted_iota(jnp.int32, (1, 128), 1)
    onehot = (z == ids).astype(jnp.float32)
    x = jnp.dot(onehot, tab_ref[...], preferred_element_type=jnp.float32)
    wa1 = wa1_ref[...]
    na = ch_ref[...] * wa1[0:1, :] + fc_ref[...] * wa1[1:2, :] + ba1_ref[...]
    na = jnp.maximum(na, 0.0)
    na = jnp.dot(na, wa2_ref[...], preferred_element_type=jnp.float32) + ba2_ref[...]
    _pack_segs(x + na, BN, out_ref)


def _upd_body(hp_ref, ap_ref, w1_ref, b1_ref, w2_ref, b2_ref, g_ref, bt_ref,
              out_ref, t_s, sum_s, ssq_s):
    b = pl.program_id(0)

    @pl.when(b < NBN)
    def _():
        sfull = _unpack(hp_ref) + _unpack(ap_ref)
        t = jnp.maximum(
            jnp.dot(sfull, w1_ref[...],
                    preferred_element_type=jnp.float32) + b1_ref[...], 0.0)
        t = (jnp.dot(t, w2_ref[...], preferred_element_type=jnp.float32)
             + b2_ref[...])
        t_s[pl.ds(b * BN, BN), :] = t

        @pl.when(b == 0)
        def _():
            sum_s[...] = jnp.zeros((1, D), jnp.float32)
            ssq_s[...] = jnp.zeros((1, D), jnp.float32)

        sum_s[...] += jnp.sum(t, axis=0, keepdims=True)
        ssq_s[...] += jnp.sum(t * t, axis=0, keepdims=True)

    @pl.when(b >= NBN)
    def _():
        t = t_s[pl.ds((b - NBN) * BN, BN), :]
        m = sum_s[...] / N
        v = ssq_s[...] / N - m * m
        y = (t - m) * lax.rsqrt(v + 1e-5) * g_ref[...] + bt_ref[...]
        y = jnp.maximum(y, 0.0)
        _pack_segs(y, BN, out_ref)


def _pool_body(hp_ref, wp_ref, bp_ref, out_ref):
    sums = [jnp.sum(hp_ref[qq], axis=0, keepdims=True) for qq in range(NSEG)]
    pooled = jnp.concatenate(sums[:NSEG - 1] + [sums[NSEG - 1][:, :HW - PAD1]],
                             axis=1) * (1.0 / N)
    out_ref[...] = (jnp.dot(pooled, wp_ref[...],
                            preferred_element_type=jnp.float32) + bp_ref[...])


def _full2d(shape):
    return pl.BlockSpec(shape, lambda *_: tuple(0 for _ in shape))


def kernel(z, chirality, formal_charge, edge_index, edge_attr, atom_table,
           Wna1, bna1, Wna2, bna2, We1, be1, We2, be2,
           Wm1, bm1, Wm2, bm2, gamma, beta, Wp, bp):
    f32 = jnp.float32

    edge_enc = pl.pallas_call(
        _edge_enc_body,
        grid=(E // BE,),
        in_specs=[
            pl.BlockSpec((BE, 3), lambda b: (b, 0)),
            _full2d((3, D)), _full2d((1, D)), _full2d((D, D)), _full2d((1, D)),
        ],
        out_specs=pl.BlockSpec((NSEG, BE, HW), lambda b: (0, b, 0)),
        out_shape=jax.ShapeDtypeStruct((NSEG, E, HW), f32),
    )
    ee = edge_enc(edge_attr, We1, be1.reshape(1, D), We2, be2.reshape(1, D))

    tab_pad = jnp.zeros((128, D), f32).at[:atom_table.shape[0]].set(atom_table)
    node_enc = pl.pallas_call(
        _node_body,
        grid=(NBN,),
        in_specs=[
            pl.BlockSpec((BN, 1), lambda b: (b, 0)),
            pl.BlockSpec((BN, 1), lambda b: (b, 0)),
            pl.BlockSpec((BN, 1), lambda b: (b, 0)),
            _full2d((128, D)), _full2d((2, D)), _full2d((1, D)),
            _full2d((D, D)), _full2d((1, D)),
        ],
        out_specs=pl.BlockSpec((NSEG, BN, HW), lambda b: (0, b, 0)),
        out_shape=jax.ShapeDtypeStruct((NSEG, N, HW), f32),
    )
    hp = node_enc(z.reshape(N, 1).astype(jnp.int32),
                  chirality.reshape(N, 1), formal_charge.reshape(N, 1),
                  tab_pad, Wna1, bna1.reshape(1, D), Wna2, bna2.reshape(1, D))

    src = edge_index[0].astype(jnp.int32)
    dst = edge_index[1].astype(jnp.int32)
    src4 = jnp.concatenate([src + qq * N for qq in range(NSEG)])
    ee_flat = ee.reshape(NSEG * E, HW)

    sc_aggr = _make_sc_aggr()

    upd = pl.pallas_call(
        _upd_body,
        grid=(2 * NBN,),
        in_specs=[
            pl.BlockSpec((NSEG, BN, HW), lambda b: (0, b % NBN, 0)),
            pl.BlockSpec((NSEG, BN, HW), lambda b: (0, b % NBN, 0)),
            _full2d((D, D)), _full2d((1, D)), _full2d((D, D)), _full2d((1, D)),
            _full2d((1, D)), _full2d((1, D)),
        ],
        out_specs=pl.BlockSpec((NSEG, BN, HW), lambda b: (0, b % NBN, 0)),
        out_shape=jax.ShapeDtypeStruct((NSEG, N, HW), f32),
        scratch_shapes=[
            pltpu.VMEM((N, D), f32),
            pltpu.VMEM((1, D), f32),
            pltpu.VMEM((1, D), f32),
        ],
    )

    for i in range(L):
        aggr = sc_aggr(hp.reshape(NSEG * N, HW), ee_flat, src4, dst)
        hp = upd(hp, aggr.reshape(NSEG, N, HW),
                 Wm1[i], bm1[i].reshape(1, D), Wm2[i], bm2[i].reshape(1, D),
                 gamma[i].reshape(1, D), beta[i].reshape(1, D))

    pool = pl.pallas_call(
        _pool_body,
        grid=(1,),
        in_specs=[pl.BlockSpec((NSEG, N, HW), lambda b: (0, 0, 0)),
                  _full2d((D, D)), _full2d((1, D))],
        out_specs=pl.BlockSpec((1, D), lambda b: (0, 0)),
        out_shape=jax.ShapeDtypeStruct((1, D), f32),
    )
    return pool(hp, Wp, bp.reshape(1, D))


# speedup vs baseline: 1.0096x; 1.0096x over previous
import functools

import jax
import jax.numpy as jnp
from jax import lax
from jax.experimental import pallas as pl
from jax.experimental.pallas import tpu as pltpu
from jax.experimental.pallas import tpu_sc as plsc

N = 10000
E = 160000
D = 300
L = 5
NSEG = 4
HW = 80
PAD1 = NSEG * HW - D

NC = 2
NS = 16
K = 80
CPT = E // NS
NCHUNK = CPT // K
STRIPE = N // NS
CPASS = 125
NPASS = STRIPE // CPASS
CPW = HW // 16
NDB = 5
NIB = 10
UNROLL = 10
NGRP = (NCHUNK + UNROLL - 1) // UNROLL


def _sc_aggr_body(h_hbm, ee_hbm, src_hbm, dst_hbm, out_hbm,
                  srcv, dstv, rows, eev, obuf, aggr_sh,
                  gsem, esem, ssem, isem):
    c = lax.axis_index("c")
    s = lax.axis_index("s")

    for p in range(2):
        q = 2 * c + p

        zv = jnp.zeros((16,), jnp.float32)

        def zrow(r, carry):
            for j in range(CPW):
                obuf[r, pl.ds(j * 16, 16)] = zv
            return carry

        lax.fori_loop(0, CPASS, zrow, 0)
        for pp in range(NPASS):
            pltpu.sync_copy(obuf,
                            aggr_sh.at[pl.ds(s * STRIPE + pp * CPASS, CPASS)])
        plsc.subcore_barrier()

        def idx_load(i, slot):
            base = s * CPT + i * K
            pltpu.async_copy(src_hbm.at[pl.ds(q * E + base, K)], srcv[slot],
                             isem[slot])
            pltpu.async_copy(dst_hbm.at[pl.ds(base, K)], dstv[slot],
                             isem[slot])

        def idx_wait(i, slot):
            base = s * CPT + i * K
            pltpu.make_async_copy(src_hbm.at[pl.ds(q * E + base, K)],
                                  srcv[slot], isem[slot]).wait()
            pltpu.make_async_copy(dst_hbm.at[pl.ds(base, K)], dstv[slot],
                                  isem[slot]).wait()

        def in_start(i, d, slot):
            pltpu.async_copy(h_hbm.at[srcv[slot]], rows[d], gsem[d])
            base = s * CPT + i * K
            pltpu.async_copy(ee_hbm.at[pl.ds(q * E + base, K)], eev[d],
                             esem[d])

        for i0 in range(5):
            idx_load(i0, i0)
        for i0 in range(3):
            idx_wait(i0, i0)
            in_start(i0, i0, i0)

        def group(g, carry):
            for j in range(UNROLL):
                i = g * UNROLL + j
                d4 = j % NDB
                d4n = (j + 3) % NDB
                i8 = j % NIB
                i8n = (j + 3) % NIB
                i8nn = (j + 5) % NIB
                live = i < NCHUNK

                @pl.when(live)
                def _():
                    pltpu.make_async_copy(h_hbm.at[srcv[i8]], rows[d4],
                                          gsem[d4]).wait()
                    base = s * CPT + i * K
                    pltpu.make_async_copy(
                        ee_hbm.at[pl.ds(q * E + base, K)], eev[d4],
                        esem[d4]).wait()

                    @plsc.parallel_loop(0, K, unroll=2)
                    def krow(k):
                        for jj in range(CPW):
                            sl = pl.ds(jj * 16, 16)
                            eev[d4][k, sl] = jnp.maximum(
                                rows[d4][k, sl] + eev[d4][k, sl], 0.0)

                    pltpu.async_copy(eev[d4], aggr_sh.at[dstv[i8]],
                                     ssem[d4], add=True)

                @pl.when((i >= 2) & (i - 2 < NCHUNK))
                def _():
                    pltpu.make_async_copy(eev[d4n],
                                          aggr_sh.at[dstv[(j + 8) % NIB]],
                                          ssem[d4n]).wait()

                @pl.when(i + 3 < NCHUNK)
                def _():
                    idx_wait(i + 3, i8n)
                    in_start(i + 3, d4n, i8n)

                @pl.when(i + 5 < NCHUNK)
                def _():
                    idx_load(i + 5, i8nn)
            return carry

        lax.fori_loop(0, NGRP, group, 0)
        plsc.subcore_barrier()

        for pp in range(NPASS):
            ro = s * STRIPE + pp * CPASS
            pltpu.sync_copy(aggr_sh.at[pl.ds(ro, CPASS)], obuf)
            pltpu.sync_copy(obuf, out_hbm.at[pl.ds(q * N + ro, CPASS)])
        plsc.subcore_barrier()


def _make_sc_aggr():
    mesh = plsc.VectorSubcoreMesh(core_axis_name="c", subcore_axis_name="s",
                                  num_cores=NC, num_subcores=NS)
    return pl.kernel(
        _sc_aggr_body,
        out_type=jax.ShapeDtypeStruct((NSEG * N, HW), jnp.float32),
        mesh=mesh,
        scratch_types=[
            [pltpu.VMEM((K,), jnp.int32) for _ in range(NIB)],
            [pltpu.VMEM((K,), jnp.int32) for _ in range(NIB)],
            [pltpu.VMEM((K, HW), jnp.float32) for _ in range(NDB)],
            [pltpu.VMEM((K, HW), jnp.float32) for _ in range(NDB)],
            pltpu.VMEM((CPASS, HW), jnp.float32),
            pltpu.VMEM_SHARED((N, HW), jnp.float32),
            [pltpu.SemaphoreType.DMA for _ in range(NDB)],
            [pltpu.SemaphoreType.DMA for _ in range(NDB)],
            [pltpu.SemaphoreType.DMA for _ in range(NDB)],
            [pltpu.SemaphoreType.DMA for _ in range(NIB)],
        ],
        compiler_params=pltpu.CompilerParams(use_tc_tiling_on_sc=False),
    )


BE = 2000
BN = 2000
NBN = N // BN


def _pack_segs(y, rows, out_ref):
    for qq in range(NSEG - 1):
        out_ref[qq] = y[:, qq * HW:(qq + 1) * HW]
    out_ref[NSEG - 1] = jnp.concatenate(
        [y[:, (NSEG - 1) * HW:], jnp.zeros((rows, PAD1), jnp.float32)], axis=1)


def _unpack(ref):
    return jnp.concatenate(
        [ref[qq] for qq in range(NSEG - 1)] + [ref[NSEG - 1][:, :HW - PAD1]],
        axis=1)


def _edge_enc_body(ea_ref, w1_ref, b1_ref, w2_ref, b2_ref, out_ref):
    ea = ea_ref[...]
    w1 = w1_ref[...]
    y = (ea[:, 0:1] * w1[0:1, :] + ea[:, 1:2] * w1[1:2, :]
         + ea[:, 2:3] * w1[2:3, :] + b1_ref[...])
    y = jnp.maximum(y, 0.0)
    y = jnp.dot(y, w2_ref[...], preferred_element_type=jnp.float32) + b2_ref[...]
    _pack_segs(y, BE, out_ref)


def _node_body(z_ref, ch_ref, fc_ref, tab_ref, wa1_ref, ba1_ref, wa2_ref,
               ba2_ref, out_ref):
    z = z_ref[...]
    ids = lax.broadcasted_iota(jnp.int32, (1, 128), 1)
    onehot = (z == ids).astype(jnp.float32)
    x = jnp.dot(onehot, tab_ref[...], preferred_element_type=jnp.float32)
    wa1 = wa1_ref[...]
    na = ch_ref[...] * wa1[0:1, :] + fc_ref[...] * wa1[1:2, :] + ba1_ref[...]
    na = jnp.maximum(na, 0.0)
    na = jnp.dot(na, wa2_ref[...], preferred_element_type=jnp.float32) + ba2_ref[...]
    _pack_segs(x + na, BN, out_ref)


def _upd1_body(hp_ref, ap_ref, w1_ref, b1_ref, w2_ref, b2_ref,
               t_ref, sum_ref, ssq_ref):
    sfull = _unpack(hp_ref) + _unpack(ap_ref)
    t = jnp.maximum(jnp.dot(sfull, w1_ref[...],
                            preferred_element_type=jnp.float32) + b1_ref[...], 0.0)
    t = jnp.dot(t, w2_ref[...], preferred_element_type=jnp.float32) + b2_ref[...]
    t_ref[...] = t
    sum_ref[0] = jnp.sum(t, axis=0, keepdims=True)
    ssq_ref[0] = jnp.sum(t * t, axis=0, keepdims=True)


def _upd2_body(t_ref, sum_ref, ssq_ref, g_ref, bt_ref, out_ref):
    t = t_ref[...]
    m = jnp.sum(sum_ref[...], axis=0) / N
    v = jnp.sum(ssq_ref[...], axis=0) / N - m * m
    y = (t - m) * lax.rsqrt(v + 1e-5) * g_ref[...] + bt_ref[...]
    y = jnp.maximum(y, 0.0)
    _pack_segs(y, BN, out_ref)


def _pool_body(hp_ref, wp_ref, bp_ref, out_ref):
    sums = [jnp.sum(hp_ref[qq], axis=0, keepdims=True) for qq in range(NSEG)]
    pooled = jnp.concatenate(sums[:NSEG - 1] + [sums[NSEG - 1][:, :HW - PAD1]],
                             axis=1) * (1.0 / N)
    out_ref[...] = (jnp.dot(pooled, wp_ref[...],
                            preferred_element_type=jnp.float32) + bp_ref[...])


def _full2d(shape):
    return pl.BlockSpec(shape, lambda *_: tuple(0 for _ in shape))


def kernel(z, chirality, formal_charge, edge_index, edge_attr, atom_table,
           Wna1, bna1, Wna2, bna2, We1, be1, We2, be2,
           Wm1, bm1, Wm2, bm2, gamma, beta, Wp, bp):
    f32 = jnp.float32

    edge_enc = pl.pallas_call(
        _edge_enc_body,
        grid=(E // BE,),
        in_specs=[
            pl.BlockSpec((BE, 3), lambda b: (b, 0)),
            _full2d((3, D)), _full2d((1, D)), _full2d((D, D)), _full2d((1, D)),
        ],
        out_specs=pl.BlockSpec((NSEG, BE, HW), lambda b: (0, b, 0)),
        out_shape=jax.ShapeDtypeStruct((NSEG, E, HW), f32),
    )
    ee = edge_enc(edge_attr, We1, be1.reshape(1, D), We2, be2.reshape(1, D))

    tab_pad = jnp.zeros((128, D), f32).at[:atom_table.shape[0]].set(atom_table)
    node_enc = pl.pallas_call(
        _node_body,
        grid=(NBN,),
        in_specs=[
            pl.BlockSpec((BN, 1), lambda b: (b, 0)),
            pl.BlockSpec((BN, 1), lambda b: (b, 0)),
            pl.BlockSpec((BN, 1), lambda b: (b, 0)),
            _full2d((128, D)), _full2d((2, D)), _full2d((1, D)),
            _full2d((D, D)), _full2d((1, D)),
        ],
        out_specs=pl.BlockSpec((NSEG, BN, HW), lambda b: (0, b, 0)),
        out_shape=jax.ShapeDtypeStruct((NSEG, N, HW), f32),
    )
    hp = node_enc(z.reshape(N, 1).astype(jnp.int32),
                  chirality.reshape(N, 1), formal_charge.reshape(N, 1),
                  tab_pad, Wna1, bna1.reshape(1, D), Wna2, bna2.reshape(1, D))

    src = edge_index[0].astype(jnp.int32)
    dst = edge_index[1].astype(jnp.int32)
    src4 = jnp.concatenate([src + qq * N for qq in range(NSEG)])
    ee_flat = ee.reshape(NSEG * E, HW)

    sc_aggr = _make_sc_aggr()

    upd1 = pl.pallas_call(
        _upd1_body,
        grid=(NBN,),
        in_specs=[
            pl.BlockSpec((NSEG, BN, HW), lambda b: (0, b, 0)),
            pl.BlockSpec((NSEG, BN, HW), lambda b: (0, b, 0)),
            _full2d((D, D)), _full2d((1, D)), _full2d((D, D)), _full2d((1, D)),
        ],
        out_specs=[
            pl.BlockSpec((BN, D), lambda b: (b, 0)),
            pl.BlockSpec((1, 1, D), lambda b: (b, 0, 0)),
            pl.BlockSpec((1, 1, D), lambda b: (b, 0, 0)),
        ],
        out_shape=[
            jax.ShapeDtypeStruct((N, D), f32),
            jax.ShapeDtypeStruct((NBN, 1, D), f32),
            jax.ShapeDtypeStruct((NBN, 1, D), f32),
        ],
    )
    upd2 = pl.pallas_call(
        _upd2_body,
        grid=(NBN,),
        in_specs=[
            pl.BlockSpec((BN, D), lambda b: (b, 0)),
            pl.BlockSpec((NBN, 1, D), lambda b: (0, 0, 0)),
            pl.BlockSpec((NBN, 1, D), lambda b: (0, 0, 0)),
            _full2d((1, D)), _full2d((1, D)),
        ],
        out_specs=pl.BlockSpec((NSEG, BN, HW), lambda b: (0, b, 0)),
        out_shape=jax.ShapeDtypeStruct((NSEG, N, HW), f32),
    )

    for i in range(L):
        aggr = sc_aggr(hp.reshape(NSEG * N, HW), ee_flat, src4, dst)
        t, sums, ssq = upd1(hp, aggr.reshape(NSEG, N, HW),
                            Wm1[i], bm1[i].reshape(1, D),
                            Wm2[i], bm2[i].reshape(1, D))
        hp = upd2(t, sums, ssq, gamma[i].reshape(1, D), beta[i].reshape(1, D))

    pool = pl.pallas_call(
        _pool_body,
        grid=(1,),
        in_specs=[pl.BlockSpec((NSEG, N, HW), lambda b: (0, 0, 0)),
                  _full2d((D, D)), _full2d((1, D))],
        out_specs=pl.BlockSpec((1, D), lambda b: (0, 0)),
        out_shape=jax.ShapeDtypeStruct((1, D), f32),
    )
    return pool(hp, Wp, bp.reshape(1, D))
